# Initial kernel scaffold; baseline (speedup 1.0000x reference)
import functools

import jax
import jax.numpy as jnp
from jax.experimental import pallas as pl
from jax.experimental.pallas import tpu as pltpu
from jax.experimental.pallas import tpu_sc as plsc

NUM_ENTITIES = 1000000
EMB_DIM = 64
BATCH = 16384

NUM_CORES = 2
NUM_SUBCORES = 16
NUM_WORKERS = NUM_CORES * NUM_SUBCORES
ROWS_PER_WORKER = BATCH // NUM_WORKERS
CHUNK = 128
CHUNKS_PER_WORKER = ROWS_PER_WORKER // CHUNK
IDX_COLS = CHUNK
IDX_ROWS = BATCH // IDX_COLS
IDX_ROWS_PER_WORKER = IDX_ROWS // NUM_WORKERS

_mesh = plsc.VectorSubcoreMesh(
    core_axis_name="c", subcore_axis_name="s",
    num_cores=NUM_CORES, num_subcores=NUM_SUBCORES)


@functools.partial(
    pl.kernel,
    out_type=(
        jax.ShapeDtypeStruct((BATCH, EMB_DIM), jnp.float32),
        jax.ShapeDtypeStruct((BATCH, EMB_DIM), jnp.float32),
        jax.ShapeDtypeStruct((BATCH, EMB_DIM), jnp.float32),
    ),
    mesh=_mesh,
    scratch_types=[
        pltpu.VMEM((IDX_ROWS_PER_WORKER, IDX_COLS), jnp.int32),
        pltpu.VMEM((IDX_ROWS_PER_WORKER, IDX_COLS), jnp.int32),
        pltpu.VMEM((IDX_ROWS_PER_WORKER, IDX_COLS), jnp.int32),
        pltpu.VMEM((ROWS_PER_WORKER, EMB_DIM), jnp.float32),
        pltpu.VMEM((ROWS_PER_WORKER, EMB_DIM), jnp.float32),
        pltpu.VMEM((ROWS_PER_WORKER, EMB_DIM), jnp.float32),
        pltpu.SemaphoreType.DMA,
    ],
)
def _sc_gather(ent_hbm, rel_hbm, idxh_hbm, idxr_hbm, idxt_hbm,
               h_out, r_out, t_out,
               idxh_v, idxr_v, idxt_v, rows_h, rows_r, rows_t, sem):
    wid = jax.lax.axis_index("s") * NUM_CORES + jax.lax.axis_index("c")
    idx_base = wid * IDX_ROWS_PER_WORKER
    pltpu.sync_copy(idxh_hbm.at[pl.ds(idx_base, IDX_ROWS_PER_WORKER)], idxh_v)
    pltpu.sync_copy(idxr_hbm.at[pl.ds(idx_base, IDX_ROWS_PER_WORKER)], idxr_v)
    pltpu.sync_copy(idxt_hbm.at[pl.ds(idx_base, IDX_ROWS_PER_WORKER)], idxt_v)
    copies = []
    for j in range(CHUNKS_PER_WORKER):
        sl = pl.ds(j * CHUNK, CHUNK)
        copies.append(pltpu.async_copy(
            ent_hbm.at[idxh_v.at[j]], rows_h.at[sl], sem))
        copies.append(pltpu.async_copy(
            rel_hbm.at[idxr_v.at[j]], rows_r.at[sl], sem))
        copies.append(pltpu.async_copy(
            ent_hbm.at[idxt_v.at[j]], rows_t.at[sl], sem))
    for c in copies:
        c.wait()
    out_base = wid * ROWS_PER_WORKER
    pltpu.sync_copy(rows_h, h_out.at[pl.ds(out_base, ROWS_PER_WORKER)])
    pltpu.sync_copy(rows_r, r_out.at[pl.ds(out_base, ROWS_PER_WORKER)])
    pltpu.sync_copy(rows_t, t_out.at[pl.ds(out_base, ROWS_PER_WORKER)])


_NORM_BLOCK = 1024


def _norm_body(idxh_ref, h_ref, idxt_ref, t_ref, ho_ref, to_ref):
    for idx_ref, x_ref, o_ref in ((idxh_ref, h_ref, ho_ref),
                                  (idxt_ref, t_ref, to_ref)):
        x = x_ref[...]
        keep = idx_ref[...] == NUM_ENTITIES - 1
        norm = jnp.sqrt(jnp.sum(x * x, axis=1, keepdims=True))
        o_ref[...] = jnp.where(keep, x, x / norm)


def _normalize(idx_h, h_raw, idx_t, t_raw):
    grid = BATCH // _NORM_BLOCK
    row_spec = pl.BlockSpec((_NORM_BLOCK, EMB_DIM), lambda i: (i, 0))
    idx_spec = pl.BlockSpec((_NORM_BLOCK, 1), lambda i: (i, 0))
    return pl.pallas_call(
        _norm_body,
        grid=(grid,),
        in_specs=[idx_spec, row_spec, idx_spec, row_spec],
        out_specs=[row_spec, row_spec],
        out_shape=[
            jax.ShapeDtypeStruct((BATCH, EMB_DIM), jnp.float32),
            jax.ShapeDtypeStruct((BATCH, EMB_DIM), jnp.float32),
        ],
    )(idx_h, h_raw, idx_t, t_raw)


def kernel(pos_h, pos_r, pos_t, entity_embds, rel_embds):
    ph = pos_h.astype(jnp.int32)
    pr = pos_r.astype(jnp.int32)
    pt = pos_t.astype(jnp.int32)
    h_raw, r_embs, t_raw = _sc_gather(
        entity_embds, rel_embds,
        ph.reshape(IDX_ROWS, IDX_COLS),
        pr.reshape(IDX_ROWS, IDX_COLS),
        pt.reshape(IDX_ROWS, IDX_COLS))
    h_embs, t_embs = _normalize(
        ph.reshape(BATCH, 1), h_raw, pt.reshape(BATCH, 1), t_raw)
    return (h_embs, t_embs)[0], r_embs, (h_embs, t_embs)[1]


# speedup vs baseline: 3.1423x; 3.1423x over previous
import functools

import jax
import jax.numpy as jnp
from jax.experimental import pallas as pl
from jax.experimental.pallas import tpu as pltpu
from jax.experimental.pallas import tpu_sc as plsc

NUM_ENTITIES = 1000000
EMB_DIM = 64
BATCH = 16384

NUM_CORES = 2
NUM_SUBCORES = 16
NUM_WORKERS = NUM_CORES * NUM_SUBCORES
ROWS_PER_WORKER = BATCH // NUM_WORKERS
CHUNK = 128
CHUNKS_PER_WORKER = ROWS_PER_WORKER // CHUNK
IDX_COLS = CHUNK
IDX_ROWS = BATCH // IDX_COLS
IDX_ROWS_PER_WORKER = IDX_ROWS // NUM_WORKERS

_mesh = plsc.VectorSubcoreMesh(
    core_axis_name="c", subcore_axis_name="s",
    num_cores=NUM_CORES, num_subcores=NUM_SUBCORES)


@functools.partial(
    pl.kernel,
    out_type=(
        jax.ShapeDtypeStruct((BATCH, EMB_DIM), jnp.float32),
        jax.ShapeDtypeStruct((BATCH, EMB_DIM), jnp.float32),
        jax.ShapeDtypeStruct((BATCH, EMB_DIM), jnp.float32),
    ),
    mesh=_mesh,
    compiler_params=pltpu.CompilerParams(use_tc_tiling_on_sc=False),
    scratch_types=[
        pltpu.VMEM((IDX_ROWS_PER_WORKER, IDX_COLS), jnp.int32),
        pltpu.VMEM((IDX_ROWS_PER_WORKER, IDX_COLS), jnp.int32),
        pltpu.VMEM((IDX_ROWS_PER_WORKER, IDX_COLS), jnp.int32),
        pltpu.VMEM((ROWS_PER_WORKER, EMB_DIM), jnp.float32),
        pltpu.VMEM((ROWS_PER_WORKER, EMB_DIM), jnp.float32),
        pltpu.VMEM((ROWS_PER_WORKER, EMB_DIM), jnp.float32),
        pltpu.SemaphoreType.DMA,
    ],
)
def _sc_gather(ent_hbm, rel_hbm, idxh_hbm, idxr_hbm, idxt_hbm,
               h_out, r_out, t_out,
               idxh_v, idxr_v, idxt_v, rows_h, rows_r, rows_t, sem):
    wid = jax.lax.axis_index("s") * NUM_CORES + jax.lax.axis_index("c")
    idx_base = wid * IDX_ROWS_PER_WORKER
    pltpu.sync_copy(idxh_hbm.at[pl.ds(idx_base, IDX_ROWS_PER_WORKER)], idxh_v)
    pltpu.sync_copy(idxr_hbm.at[pl.ds(idx_base, IDX_ROWS_PER_WORKER)], idxr_v)
    pltpu.sync_copy(idxt_hbm.at[pl.ds(idx_base, IDX_ROWS_PER_WORKER)], idxt_v)
    copies = []
    for j in range(CHUNKS_PER_WORKER):
        sl = pl.ds(j * CHUNK, CHUNK)
        copies.append(pltpu.async_copy(
            ent_hbm.at[idxh_v.at[j]], rows_h.at[sl], sem))
        copies.append(pltpu.async_copy(
            rel_hbm.at[idxr_v.at[j]], rows_r.at[sl], sem))
        copies.append(pltpu.async_copy(
            ent_hbm.at[idxt_v.at[j]], rows_t.at[sl], sem))
    for c in copies:
        c.wait()
    out_base = wid * ROWS_PER_WORKER
    pltpu.sync_copy(rows_h, h_out.at[pl.ds(out_base, ROWS_PER_WORKER)])
    pltpu.sync_copy(rows_r, r_out.at[pl.ds(out_base, ROWS_PER_WORKER)])
    pltpu.sync_copy(rows_t, t_out.at[pl.ds(out_base, ROWS_PER_WORKER)])


_NORM_BLOCK = 1024


def _norm_body(idxh_ref, h_ref, idxt_ref, t_ref, ho_ref, to_ref):
    for idx_ref, x_ref, o_ref in ((idxh_ref, h_ref, ho_ref),
                                  (idxt_ref, t_ref, to_ref)):
        x = x_ref[...]
        keep = idx_ref[...] == NUM_ENTITIES - 1
        norm = jnp.sqrt(jnp.sum(x * x, axis=1, keepdims=True))
        o_ref[...] = jnp.where(keep, x, x / norm)


def _normalize(idx_h, h_raw, idx_t, t_raw):
    grid = BATCH // _NORM_BLOCK
    row_spec = pl.BlockSpec((_NORM_BLOCK, EMB_DIM), lambda i: (i, 0))
    idx_spec = pl.BlockSpec((_NORM_BLOCK, 1), lambda i: (i, 0))
    return pl.pallas_call(
        _norm_body,
        grid=(grid,),
        in_specs=[idx_spec, row_spec, idx_spec, row_spec],
        out_specs=[row_spec, row_spec],
        out_shape=[
            jax.ShapeDtypeStruct((BATCH, EMB_DIM), jnp.float32),
            jax.ShapeDtypeStruct((BATCH, EMB_DIM), jnp.float32),
        ],
    )(idx_h, h_raw, idx_t, t_raw)


def kernel(pos_h, pos_r, pos_t, entity_embds, rel_embds):
    ph = pos_h.astype(jnp.int32)
    pr = pos_r.astype(jnp.int32)
    pt = pos_t.astype(jnp.int32)
    h_raw, r_embs, t_raw = _sc_gather(
        entity_embds, rel_embds,
        ph.reshape(IDX_ROWS, IDX_COLS),
        pr.reshape(IDX_ROWS, IDX_COLS),
        pt.reshape(IDX_ROWS, IDX_COLS))
    h_embs, t_embs = _normalize(
        ph.reshape(BATCH, 1), h_raw, pt.reshape(BATCH, 1), t_raw)
    return (h_embs, r_embs, t_embs)


# speedup vs baseline: 4.3104x; 1.3718x over previous
import functools

import jax
import jax.numpy as jnp
from jax.experimental import pallas as pl
from jax.experimental.pallas import tpu as pltpu
from jax.experimental.pallas import tpu_sc as plsc

NUM_ENTITIES = 1000000
NUM_RELATIONS = 1000
EMB_DIM = 64
BATCH = 16384

NUM_CORES = 2
NUM_SUBCORES = 16
NUM_WORKERS = NUM_CORES * NUM_SUBCORES
ROWS_PER_WORKER = BATCH // NUM_WORKERS
CHUNK = 128
CHUNKS_PER_WORKER = ROWS_PER_WORKER // CHUNK

_REPACK_W = 8192


def _repack_body(x_ref, o_ref):
    x = x_ref[...]
    xt = jnp.swapaxes(x, 0, 1)
    x3 = xt.reshape(_REPACK_W // 2, 2, EMB_DIM)
    o_ref[:, :EMB_DIM] = x3[:, 0, :]
    o_ref[:, EMB_DIM:] = x3[:, 1, :]


def _repack(tbl_t, n_rows):
    n_packed = (n_rows + 1) // 2
    grid = (n_rows + _REPACK_W - 1) // _REPACK_W
    return pl.pallas_call(
        _repack_body,
        grid=(grid,),
        in_specs=[pl.BlockSpec((EMB_DIM, _REPACK_W), lambda i: (0, i))],
        out_specs=pl.BlockSpec((_REPACK_W // 2, 2 * EMB_DIM), lambda i: (i, 0)),
        out_shape=jax.ShapeDtypeStruct((n_packed, 2 * EMB_DIM), jnp.float32),
    )(tbl_t)


_mesh = plsc.VectorSubcoreMesh(
    core_axis_name="c", subcore_axis_name="s",
    num_cores=NUM_CORES, num_subcores=NUM_SUBCORES)


@functools.partial(
    pl.kernel,
    out_type=(
        jax.ShapeDtypeStruct((BATCH, 2 * EMB_DIM), jnp.float32),
        jax.ShapeDtypeStruct((BATCH, 2 * EMB_DIM), jnp.float32),
        jax.ShapeDtypeStruct((BATCH, 2 * EMB_DIM), jnp.float32),
    ),
    mesh=_mesh,
    compiler_params=pltpu.CompilerParams(use_tc_tiling_on_sc=True),
    scratch_types=[
        pltpu.VMEM((ROWS_PER_WORKER,), jnp.int32),
        pltpu.VMEM((ROWS_PER_WORKER, 2 * EMB_DIM), jnp.float32),
        pltpu.SemaphoreType.DMA,
        pltpu.SemaphoreType.DMA,
    ],
)
def _sc_gather(ent_p, rel_p, idxh_hbm, idxr_hbm, idxt_hbm,
               h_out, r_out, t_out,
               idx_v, rows_v, gsem, osem):
    wid = jax.lax.axis_index("s") * NUM_CORES + jax.lax.axis_index("c")
    base = wid * ROWS_PER_WORKER
    for idx_hbm, tbl, out in ((idxh_hbm, ent_p, h_out),
                              (idxr_hbm, rel_p, r_out),
                              (idxt_hbm, ent_p, t_out)):
        pltpu.sync_copy(idx_hbm.at[pl.ds(base, ROWS_PER_WORKER)], idx_v)
        for k in range(ROWS_PER_WORKER // 16):
            sl = pl.ds(k * 16, 16)
            idx_v[sl] = jax.lax.shift_right_logical(idx_v[sl], 1)
        copies = []
        for j in range(CHUNKS_PER_WORKER):
            sl = pl.ds(j * CHUNK, CHUNK)
            copies.append(pltpu.async_copy(
                tbl.at[idx_v.at[sl]], rows_v.at[sl], gsem))
        for c in copies:
            c.wait()
        pltpu.async_copy(
            rows_v, out.at[pl.ds(base, ROWS_PER_WORKER)], osem).wait()


_NORM_BLOCK = 1024


def _half(x, idx):
    par = (idx & 1) == 1
    return jnp.where(par, x[:, EMB_DIM:], x[:, :EMB_DIM])


def _norm_body(idxh_ref, h_ref, idxt_ref, t_ref, idxr_ref, r_ref,
               ho_ref, to_ref, ro_ref):
    idxr = idxr_ref[...]
    ro_ref[...] = _half(r_ref[...], idxr)
    for idx_ref, x_ref, o_ref in ((idxh_ref, h_ref, ho_ref),
                                  (idxt_ref, t_ref, to_ref)):
        idx = idx_ref[...]
        v = _half(x_ref[...], idx)
        keep = idx == NUM_ENTITIES - 1
        norm = jnp.sqrt(jnp.sum(v * v, axis=1, keepdims=True))
        o_ref[...] = jnp.where(keep, v, v / norm)


def _normalize(idx_h, h_p, idx_t, t_p, idx_r, r_p):
    grid = BATCH // _NORM_BLOCK
    pair_spec = pl.BlockSpec((_NORM_BLOCK, 2 * EMB_DIM), lambda i: (i, 0))
    out_spec = pl.BlockSpec((_NORM_BLOCK, EMB_DIM), lambda i: (i, 0))
    idx_spec = pl.BlockSpec((_NORM_BLOCK, 1), lambda i: (i, 0))
    return pl.pallas_call(
        _norm_body,
        grid=(grid,),
        in_specs=[idx_spec, pair_spec, idx_spec, pair_spec,
                  idx_spec, pair_spec],
        out_specs=[out_spec, out_spec, out_spec],
        out_shape=[
            jax.ShapeDtypeStruct((BATCH, EMB_DIM), jnp.float32),
            jax.ShapeDtypeStruct((BATCH, EMB_DIM), jnp.float32),
            jax.ShapeDtypeStruct((BATCH, EMB_DIM), jnp.float32),
        ],
    )(idx_h, h_p, idx_t, t_p, idx_r, r_p)


def kernel(pos_h, pos_r, pos_t, entity_embds, rel_embds):
    ph = pos_h.astype(jnp.int32)
    pr = pos_r.astype(jnp.int32)
    pt = pos_t.astype(jnp.int32)
    ent_p = _repack(jnp.swapaxes(entity_embds, 0, 1), NUM_ENTITIES)
    rel_p = _repack(jnp.swapaxes(rel_embds, 0, 1), NUM_RELATIONS)
    h_p, r_p, t_p = _sc_gather(ent_p, rel_p, ph, pr, pt)
    h_embs, t_embs, r_embs = _normalize(
        ph.reshape(BATCH, 1), h_p, pt.reshape(BATCH, 1), t_p,
        pr.reshape(BATCH, 1), r_p)
    return (h_embs, r_embs, t_embs)


# speedup vs baseline: 6.2992x; 1.4614x over previous
import functools

import jax
import jax.numpy as jnp
from jax.experimental import pallas as pl
from jax.experimental.pallas import tpu as pltpu
from jax.experimental.pallas import tpu_sc as plsc

NUM_ENTITIES = 1000000
NUM_RELATIONS = 1000
EMB_DIM = 64
BATCH = 16384

NUM_CORES = 2
NUM_SUBCORES = 16
NUM_WORKERS = NUM_CORES * NUM_SUBCORES
ROWS_PER_WORKER = BATCH // NUM_WORKERS
CHUNK = 128
CHUNKS_PER_WORKER = ROWS_PER_WORKER // CHUNK

_REPACK_W = 8192


def _repack_body(x_ref, o_ref):
    sel = jax.lax.broadcasted_iota(jnp.int32, (EMB_DIM, 2 * EMB_DIM), 0)
    tgt = jax.lax.broadcasted_iota(jnp.int32, (EMB_DIM, 2 * EMB_DIM), 1)
    s_even = (tgt == 2 * sel).astype(jnp.float32)
    s_odd = (tgt == 2 * sel + 1).astype(jnp.float32)
    dn = (((1,), (1,)), ((), ()))
    for c in range(_REPACK_W // (2 * EMB_DIM)):
        x = x_ref[:, pl.ds(c * 2 * EMB_DIM, 2 * EMB_DIM)]
        even = jax.lax.dot_general(
            s_even, x, dn, preferred_element_type=jnp.float32)
        odd = jax.lax.dot_general(
            s_odd, x, dn, preferred_element_type=jnp.float32)
        o_ref[pl.ds(c * EMB_DIM, EMB_DIM), :EMB_DIM] = even
        o_ref[pl.ds(c * EMB_DIM, EMB_DIM), EMB_DIM:] = odd


def _repack(tbl_t, n_rows):
    n_packed = (n_rows + 1) // 2
    grid = (n_rows + _REPACK_W - 1) // _REPACK_W
    return pl.pallas_call(
        _repack_body,
        grid=(grid,),
        in_specs=[pl.BlockSpec((EMB_DIM, _REPACK_W), lambda i: (0, i))],
        out_specs=pl.BlockSpec((_REPACK_W // 2, 2 * EMB_DIM), lambda i: (i, 0)),
        out_shape=jax.ShapeDtypeStruct((n_packed, 2 * EMB_DIM), jnp.float32),
    )(tbl_t)


_mesh = plsc.VectorSubcoreMesh(
    core_axis_name="c", subcore_axis_name="s",
    num_cores=NUM_CORES, num_subcores=NUM_SUBCORES)


@functools.partial(
    pl.kernel,
    out_type=(
        jax.ShapeDtypeStruct((BATCH, 2 * EMB_DIM), jnp.float32),
        jax.ShapeDtypeStruct((BATCH, 2 * EMB_DIM), jnp.float32),
        jax.ShapeDtypeStruct((BATCH, 2 * EMB_DIM), jnp.float32),
    ),
    mesh=_mesh,
    compiler_params=pltpu.CompilerParams(use_tc_tiling_on_sc=True),
    scratch_types=[
        pltpu.VMEM((ROWS_PER_WORKER,), jnp.int32),
        pltpu.VMEM((ROWS_PER_WORKER, 2 * EMB_DIM), jnp.float32),
        pltpu.SemaphoreType.DMA,
        pltpu.SemaphoreType.DMA,
    ],
)
def _sc_gather(ent_p, rel_p, idxh_hbm, idxr_hbm, idxt_hbm,
               h_out, r_out, t_out,
               idx_v, rows_v, gsem, osem):
    wid = jax.lax.axis_index("s") * NUM_CORES + jax.lax.axis_index("c")
    base = wid * ROWS_PER_WORKER
    for idx_hbm, tbl, out in ((idxh_hbm, ent_p, h_out),
                              (idxr_hbm, rel_p, r_out),
                              (idxt_hbm, ent_p, t_out)):
        pltpu.sync_copy(idx_hbm.at[pl.ds(base, ROWS_PER_WORKER)], idx_v)
        for k in range(ROWS_PER_WORKER // 16):
            sl = pl.ds(k * 16, 16)
            idx_v[sl] = jax.lax.shift_right_logical(idx_v[sl], 1)
        copies = []
        for j in range(CHUNKS_PER_WORKER):
            sl = pl.ds(j * CHUNK, CHUNK)
            copies.append(pltpu.async_copy(
                tbl.at[idx_v.at[sl]], rows_v.at[sl], gsem))
        for c in copies:
            c.wait()
        pltpu.async_copy(
            rows_v, out.at[pl.ds(base, ROWS_PER_WORKER)], osem).wait()


_NORM_BLOCK = 1024


def _half(x, idx):
    par = (idx & 1) == 1
    return jnp.where(par, x[:, EMB_DIM:], x[:, :EMB_DIM])


def _norm_body(idxh_ref, h_ref, idxt_ref, t_ref, idxr_ref, r_ref,
               ho_ref, to_ref, ro_ref):
    idxr = idxr_ref[...]
    ro_ref[...] = _half(r_ref[...], idxr)
    for idx_ref, x_ref, o_ref in ((idxh_ref, h_ref, ho_ref),
                                  (idxt_ref, t_ref, to_ref)):
        idx = idx_ref[...]
        v = _half(x_ref[...], idx)
        keep = idx == NUM_ENTITIES - 1
        norm = jnp.sqrt(jnp.sum(v * v, axis=1, keepdims=True))
        o_ref[...] = jnp.where(keep, v, v / norm)


def _normalize(idx_h, h_p, idx_t, t_p, idx_r, r_p):
    grid = BATCH // _NORM_BLOCK
    pair_spec = pl.BlockSpec((_NORM_BLOCK, 2 * EMB_DIM), lambda i: (i, 0))
    out_spec = pl.BlockSpec((_NORM_BLOCK, EMB_DIM), lambda i: (i, 0))
    idx_spec = pl.BlockSpec((_NORM_BLOCK, 1), lambda i: (i, 0))
    return pl.pallas_call(
        _norm_body,
        grid=(grid,),
        in_specs=[idx_spec, pair_spec, idx_spec, pair_spec,
                  idx_spec, pair_spec],
        out_specs=[out_spec, out_spec, out_spec],
        out_shape=[
            jax.ShapeDtypeStruct((BATCH, EMB_DIM), jnp.float32),
            jax.ShapeDtypeStruct((BATCH, EMB_DIM), jnp.float32),
            jax.ShapeDtypeStruct((BATCH, EMB_DIM), jnp.float32),
        ],
    )(idx_h, h_p, idx_t, t_p, idx_r, r_p)


def kernel(pos_h, pos_r, pos_t, entity_embds, rel_embds):
    ph = pos_h.astype(jnp.int32)
    pr = pos_r.astype(jnp.int32)
    pt = pos_t.astype(jnp.int32)
    ent_p = _repack(jnp.swapaxes(entity_embds, 0, 1), NUM_ENTITIES)
    rel_p = _repack(jnp.swapaxes(rel_embds, 0, 1), NUM_RELATIONS)
    h_p, r_p, t_p = _sc_gather(ent_p, rel_p, ph, pr, pt)
    h_embs, t_embs, r_embs = _normalize(
        ph.reshape(BATCH, 1), h_p, pt.reshape(BATCH, 1), t_p,
        pr.reshape(BATCH, 1), r_p)
    return (h_embs, r_embs, t_embs)


# speedup vs baseline: 6.7267x; 1.0679x over previous
import functools

import jax
import jax.numpy as jnp
from jax.experimental import pallas as pl
from jax.experimental.pallas import tpu as pltpu
from jax.experimental.pallas import tpu_sc as plsc

NUM_ENTITIES = 1000000
NUM_RELATIONS = 1000
EMB_DIM = 64
BATCH = 16384

NUM_CORES = 2
NUM_SUBCORES = 16
NUM_WORKERS = NUM_CORES * NUM_SUBCORES
ROWS_PER_WORKER = BATCH // NUM_WORKERS
CHUNK = 128
CHUNKS_PER_WORKER = ROWS_PER_WORKER // CHUNK

_REPACK_W = 8192


def _repack_body(x_ref, o_ref):
    sel = jax.lax.broadcasted_iota(jnp.int32, (EMB_DIM, 2 * EMB_DIM), 0)
    tgt = jax.lax.broadcasted_iota(jnp.int32, (EMB_DIM, 2 * EMB_DIM), 1)
    s_even = (tgt == 2 * sel).astype(jnp.float32)
    s_odd = (tgt == 2 * sel + 1).astype(jnp.float32)
    dn = (((1,), (1,)), ((), ()))
    for c in range(_REPACK_W // (2 * EMB_DIM)):
        x = x_ref[:, pl.ds(c * 2 * EMB_DIM, 2 * EMB_DIM)]
        even = jax.lax.dot_general(
            s_even, x, dn, preferred_element_type=jnp.float32)
        odd = jax.lax.dot_general(
            s_odd, x, dn, preferred_element_type=jnp.float32)
        o_ref[pl.ds(c * EMB_DIM, EMB_DIM), :EMB_DIM] = even
        o_ref[pl.ds(c * EMB_DIM, EMB_DIM), EMB_DIM:] = odd


def _repack(tbl_t, n_rows):
    n_packed = (n_rows + 1) // 2
    grid = (n_rows + _REPACK_W - 1) // _REPACK_W
    return pl.pallas_call(
        _repack_body,
        grid=(grid,),
        in_specs=[pl.BlockSpec((EMB_DIM, _REPACK_W), lambda i: (0, i))],
        out_specs=pl.BlockSpec((_REPACK_W // 2, 2 * EMB_DIM), lambda i: (i, 0)),
        out_shape=jax.ShapeDtypeStruct((n_packed, 2 * EMB_DIM), jnp.float32),
    )(tbl_t)


_mesh = plsc.VectorSubcoreMesh(
    core_axis_name="c", subcore_axis_name="s",
    num_cores=NUM_CORES, num_subcores=NUM_SUBCORES)


@functools.partial(
    pl.kernel,
    out_type=(
        jax.ShapeDtypeStruct((BATCH, 2 * EMB_DIM), jnp.float32),
        jax.ShapeDtypeStruct((BATCH, 2 * EMB_DIM), jnp.float32),
        jax.ShapeDtypeStruct((BATCH, 2 * EMB_DIM), jnp.float32),
    ),
    mesh=_mesh,
    compiler_params=pltpu.CompilerParams(use_tc_tiling_on_sc=True),
    scratch_types=[
        pltpu.VMEM((ROWS_PER_WORKER,), jnp.int32),
        pltpu.VMEM((ROWS_PER_WORKER, 2 * EMB_DIM), jnp.float32),
        pltpu.SemaphoreType.DMA,
        pltpu.SemaphoreType.DMA,
    ],
)
def _sc_gather(ent_p, rel_p, idxh_hbm, idxr_hbm, idxt_hbm,
               h_out, r_out, t_out,
               idx_v, rows_v, gsem, osem):
    wid = jax.lax.axis_index("s") * NUM_CORES + jax.lax.axis_index("c")
    base = wid * ROWS_PER_WORKER
    for idx_hbm, tbl, out in ((idxh_hbm, ent_p, h_out),
                              (idxr_hbm, rel_p, r_out),
                              (idxt_hbm, ent_p, t_out)):
        pltpu.sync_copy(idx_hbm.at[pl.ds(base, ROWS_PER_WORKER)], idx_v)
        for k in range(ROWS_PER_WORKER // 16):
            sl = pl.ds(k * 16, 16)
            idx_v[sl] = jax.lax.shift_right_logical(idx_v[sl], 1)
        copies = []
        for j in range(CHUNKS_PER_WORKER):
            sl = pl.ds(j * CHUNK, CHUNK)
            copies.append(pltpu.async_copy(
                tbl.at[idx_v.at[sl]], rows_v.at[sl], gsem))
        for c in copies:
            c.wait()
        pltpu.async_copy(
            rows_v, out.at[pl.ds(base, ROWS_PER_WORKER)], osem).wait()


_NORM_BLOCK = 1024


def _half(x, idx):
    par = (idx & 1) == 1
    return jnp.where(par, x[:, EMB_DIM:], x[:, :EMB_DIM])


def _mxu_t(v):
    eye = jnp.eye(EMB_DIM, dtype=jnp.float32)
    return jax.lax.dot_general(
        eye, v, (((1,), (1,)), ((), ())), preferred_element_type=jnp.float32)


def _norm_body(idxh_ref, h_ref, idxt_ref, t_ref, idxr_ref, r_ref,
               ho_ref, to_ref, ro_ref):
    idxr = idxr_ref[...]
    ro_ref[...] = _mxu_t(_half(r_ref[...], idxr))
    for idx_ref, x_ref, o_ref in ((idxh_ref, h_ref, ho_ref),
                                  (idxt_ref, t_ref, to_ref)):
        idx = idx_ref[...]
        v = _half(x_ref[...], idx)
        keep = idx == NUM_ENTITIES - 1
        norm = jnp.sqrt(jnp.sum(v * v, axis=1, keepdims=True))
        o_ref[...] = _mxu_t(jnp.where(keep, v, v / norm))


def _normalize(idx_h, h_p, idx_t, t_p, idx_r, r_p):
    grid = BATCH // _NORM_BLOCK
    pair_spec = pl.BlockSpec((_NORM_BLOCK, 2 * EMB_DIM), lambda i: (i, 0))
    out_spec = pl.BlockSpec((EMB_DIM, _NORM_BLOCK), lambda i: (0, i))
    idx_spec = pl.BlockSpec((_NORM_BLOCK, 1), lambda i: (i, 0))
    return pl.pallas_call(
        _norm_body,
        grid=(grid,),
        in_specs=[idx_spec, pair_spec, idx_spec, pair_spec,
                  idx_spec, pair_spec],
        out_specs=[out_spec, out_spec, out_spec],
        out_shape=[
            jax.ShapeDtypeStruct((EMB_DIM, BATCH), jnp.float32),
            jax.ShapeDtypeStruct((EMB_DIM, BATCH), jnp.float32),
            jax.ShapeDtypeStruct((EMB_DIM, BATCH), jnp.float32),
        ],
    )(idx_h, h_p, idx_t, t_p, idx_r, r_p)


def kernel(pos_h, pos_r, pos_t, entity_embds, rel_embds):
    ph = pos_h.astype(jnp.int32)
    pr = pos_r.astype(jnp.int32)
    pt = pos_t.astype(jnp.int32)
    ent_p = _repack(jnp.swapaxes(entity_embds, 0, 1), NUM_ENTITIES)
    rel_p = _repack(jnp.swapaxes(rel_embds, 0, 1), NUM_RELATIONS)
    h_p, r_p, t_p = _sc_gather(ent_p, rel_p, ph, pr, pt)
    h_t, t_t, r_t = _normalize(
        ph.reshape(BATCH, 1), h_p, pt.reshape(BATCH, 1), t_p,
        pr.reshape(BATCH, 1), r_p)
    return (jnp.swapaxes(h_t, 0, 1),
            jnp.swapaxes(r_t, 0, 1),
            jnp.swapaxes(t_t, 0, 1))


# speedup vs baseline: 6.7721x; 1.0068x over previous
import functools

import jax
import jax.numpy as jnp
from jax.experimental import pallas as pl
from jax.experimental.pallas import tpu as pltpu
from jax.experimental.pallas import tpu_sc as plsc

NUM_ENTITIES = 1000000
NUM_RELATIONS = 1000
EMB_DIM = 64
BATCH = 16384

NUM_CORES = 2
NUM_SUBCORES = 16
NUM_WORKERS = NUM_CORES * NUM_SUBCORES
ROWS_PER_WORKER = BATCH // NUM_WORKERS
CHUNK = 128
CHUNKS_PER_WORKER = ROWS_PER_WORKER // CHUNK

_REPACK_W = 8192


def _repack_body(x_ref, o_ref):
    sel = jax.lax.broadcasted_iota(jnp.int32, (EMB_DIM, 2 * EMB_DIM), 0)
    tgt = jax.lax.broadcasted_iota(jnp.int32, (EMB_DIM, 2 * EMB_DIM), 1)
    s_even = (tgt == 2 * sel).astype(jnp.float32)
    s_odd = (tgt == 2 * sel + 1).astype(jnp.float32)
    dn = (((1,), (1,)), ((), ()))
    for c in range(_REPACK_W // (2 * EMB_DIM)):
        x = x_ref[:, pl.ds(c * 2 * EMB_DIM, 2 * EMB_DIM)]
        even = jax.lax.dot_general(
            s_even, x, dn, preferred_element_type=jnp.float32)
        odd = jax.lax.dot_general(
            s_odd, x, dn, preferred_element_type=jnp.float32)
        o_ref[pl.ds(c * EMB_DIM, EMB_DIM), :EMB_DIM] = even
        o_ref[pl.ds(c * EMB_DIM, EMB_DIM), EMB_DIM:] = odd


def _repack(tbl_t, n_rows):
    n_packed = (n_rows + 1) // 2
    grid = (n_rows + _REPACK_W - 1) // _REPACK_W
    return pl.pallas_call(
        _repack_body,
        grid=(grid,),
        in_specs=[pl.BlockSpec((EMB_DIM, _REPACK_W), lambda i: (0, i))],
        out_specs=pl.BlockSpec((_REPACK_W // 2, 2 * EMB_DIM), lambda i: (i, 0)),
        out_shape=jax.ShapeDtypeStruct((n_packed, 2 * EMB_DIM), jnp.float32),
    )(tbl_t)


_mesh = plsc.VectorSubcoreMesh(
    core_axis_name="c", subcore_axis_name="s",
    num_cores=NUM_CORES, num_subcores=NUM_SUBCORES)


@functools.partial(
    pl.kernel,
    out_type=(
        jax.ShapeDtypeStruct((BATCH, 2 * EMB_DIM), jnp.float32),
        jax.ShapeDtypeStruct((BATCH, 2 * EMB_DIM), jnp.float32),
        jax.ShapeDtypeStruct((BATCH, 2 * EMB_DIM), jnp.float32),
    ),
    mesh=_mesh,
    compiler_params=pltpu.CompilerParams(use_tc_tiling_on_sc=True),
    scratch_types=[
        pltpu.VMEM((ROWS_PER_WORKER,), jnp.int32),
        pltpu.VMEM((ROWS_PER_WORKER,), jnp.int32),
        pltpu.VMEM((ROWS_PER_WORKER,), jnp.int32),
        pltpu.VMEM((ROWS_PER_WORKER, 2 * EMB_DIM), jnp.float32),
        pltpu.SemaphoreType.DMA,
        pltpu.SemaphoreType.DMA,
    ],
)
def _sc_gather(ent_p, rel_p, idxh_hbm, idxr_hbm, idxt_hbm,
               h_out, r_out, t_out,
               idxh_v, idxr_v, idxt_v, rows_v, gsem, osem):
    wid = jax.lax.axis_index("s") * NUM_CORES + jax.lax.axis_index("c")
    base = wid * ROWS_PER_WORKER
    jobs = ((idxh_hbm, ent_p, h_out, idxh_v),
            (idxr_hbm, rel_p, r_out, idxr_v),
            (idxt_hbm, ent_p, t_out, idxt_v))
    for idx_hbm, _, _, idx_v in jobs:
        pltpu.sync_copy(idx_hbm.at[pl.ds(base, ROWS_PER_WORKER)], idx_v)
        for k in range(ROWS_PER_WORKER // 16):
            sl = pl.ds(k * 16, 16)
            idx_v[sl] = jax.lax.shift_right_logical(idx_v[sl], 1)
    chunk_jobs = [(t, j) for t in range(len(jobs))
                  for j in range(CHUNKS_PER_WORKER)]
    depth = CHUNKS_PER_WORKER
    n = len(chunk_jobs)
    g = [None] * n
    o = [None] * n

    def _issue_out(k):
        t, j = chunk_jobs[k]
        out = jobs[t][2]
        ssl = pl.ds((k % depth) * CHUNK, CHUNK)
        o[k] = pltpu.async_copy(
            rows_v.at[ssl], out.at[pl.ds(base + j * CHUNK, CHUNK)], osem)

    for k in range(n):
        t, j = chunk_jobs[k]
        tbl, idx_v = jobs[t][1], jobs[t][3]
        if k >= depth:
            o[k - depth].wait()
        g[k] = pltpu.async_copy(
            tbl.at[idx_v.at[pl.ds(j * CHUNK, CHUNK)]],
            rows_v.at[pl.ds((k % depth) * CHUNK, CHUNK)], gsem)
        if k >= 1:
            g[k - 1].wait()
            _issue_out(k - 1)
    g[n - 1].wait()
    _issue_out(n - 1)
    for k in range(n - depth, n):
        o[k].wait()


_NORM_BLOCK = 1024


def _half(x, idx):
    par = (idx & 1) == 1
    return jnp.where(par, x[:, EMB_DIM:], x[:, :EMB_DIM])


def _mxu_t(v):
    eye = jnp.eye(EMB_DIM, dtype=jnp.float32)
    return jax.lax.dot_general(
        eye, v, (((1,), (1,)), ((), ())), preferred_element_type=jnp.float32)


def _norm_body(idxh_ref, h_ref, idxt_ref, t_ref, idxr_ref, r_ref,
               ho_ref, to_ref, ro_ref):
    idxr = idxr_ref[...]
    ro_ref[...] = _mxu_t(_half(r_ref[...], idxr))
    for idx_ref, x_ref, o_ref in ((idxh_ref, h_ref, ho_ref),
                                  (idxt_ref, t_ref, to_ref)):
        idx = idx_ref[...]
        v = _half(x_ref[...], idx)
        keep = idx == NUM_ENTITIES - 1
        norm = jnp.sqrt(jnp.sum(v * v, axis=1, keepdims=True))
        o_ref[...] = _mxu_t(jnp.where(keep, v, v / norm))


def _normalize(idx_h, h_p, idx_t, t_p, idx_r, r_p):
    grid = BATCH // _NORM_BLOCK
    pair_spec = pl.BlockSpec((_NORM_BLOCK, 2 * EMB_DIM), lambda i: (i, 0))
    out_spec = pl.BlockSpec((EMB_DIM, _NORM_BLOCK), lambda i: (0, i))
    idx_spec = pl.BlockSpec((_NORM_BLOCK, 1), lambda i: (i, 0))
    return pl.pallas_call(
        _norm_body,
        grid=(grid,),
        in_specs=[idx_spec, pair_spec, idx_spec, pair_spec,
                  idx_spec, pair_spec],
        out_specs=[out_spec, out_spec, out_spec],
        out_shape=[
            jax.ShapeDtypeStruct((EMB_DIM, BATCH), jnp.float32),
            jax.ShapeDtypeStruct((EMB_DIM, BATCH), jnp.float32),
            jax.ShapeDtypeStruct((EMB_DIM, BATCH), jnp.float32),
        ],
    )(idx_h, h_p, idx_t, t_p, idx_r, r_p)


def kernel(pos_h, pos_r, pos_t, entity_embds, rel_embds):
    ph = pos_h.astype(jnp.int32)
    pr = pos_r.astype(jnp.int32)
    pt = pos_t.astype(jnp.int32)
    ent_p = _repack(jnp.swapaxes(entity_embds, 0, 1), NUM_ENTITIES)
    rel_p = _repack(jnp.swapaxes(rel_embds, 0, 1), NUM_RELATIONS)
    h_p, r_p, t_p = _sc_gather(ent_p, rel_p, ph, pr, pt)
    h_t, t_t, r_t = _normalize(
        ph.reshape(BATCH, 1), h_p, pt.reshape(BATCH, 1), t_p,
        pr.reshape(BATCH, 1), r_p)
    return (jnp.swapaxes(h_t, 0, 1),
            jnp.swapaxes(r_t, 0, 1),
            jnp.swapaxes(t_t, 0, 1))


# speedup vs baseline: 8.1736x; 1.2070x over previous
import functools

import jax
import jax.numpy as jnp
from jax.experimental import pallas as pl
from jax.experimental.pallas import tpu as pltpu
from jax.experimental.pallas import tpu_sc as plsc

NUM_ENTITIES = 1000000
NUM_RELATIONS = 1000
EMB_DIM = 64
BATCH = 16384

NUM_CORES = 2
NUM_SUBCORES = 16
NUM_WORKERS = NUM_CORES * NUM_SUBCORES
ROWS_PER_WORKER = BATCH // NUM_WORKERS
CHUNK = 128
CHUNKS_PER_WORKER = ROWS_PER_WORKER // CHUNK

_REPACK_W = 32768


def _repack_body(x_ref, o_ref):
    sel = jax.lax.broadcasted_iota(jnp.int32, (EMB_DIM, 2 * EMB_DIM), 0)
    tgt = jax.lax.broadcasted_iota(jnp.int32, (EMB_DIM, 2 * EMB_DIM), 1)
    s_even = (tgt == 2 * sel).astype(jnp.float32)
    s_odd = (tgt == 2 * sel + 1).astype(jnp.float32)
    dn = (((1,), (1,)), ((), ()))
    for c in range(_REPACK_W // (2 * EMB_DIM)):
        x = x_ref[:, pl.ds(c * 2 * EMB_DIM, 2 * EMB_DIM)]
        even = jax.lax.dot_general(
            s_even, x, dn, preferred_element_type=jnp.float32)
        odd = jax.lax.dot_general(
            s_odd, x, dn, preferred_element_type=jnp.float32)
        o_ref[pl.ds(c * EMB_DIM, EMB_DIM), :EMB_DIM] = even
        o_ref[pl.ds(c * EMB_DIM, EMB_DIM), EMB_DIM:] = odd


def _repack(tbl_t, n_rows):
    n_packed = (n_rows + 1) // 2
    grid = (n_rows + _REPACK_W - 1) // _REPACK_W
    return pl.pallas_call(
        _repack_body,
        grid=(grid,),
        in_specs=[pl.BlockSpec((EMB_DIM, _REPACK_W), lambda i: (0, i))],
        out_specs=pl.BlockSpec((_REPACK_W // 2, 2 * EMB_DIM), lambda i: (i, 0)),
        out_shape=jax.ShapeDtypeStruct((n_packed, 2 * EMB_DIM), jnp.float32),
    )(tbl_t)


_mesh = plsc.VectorSubcoreMesh(
    core_axis_name="c", subcore_axis_name="s",
    num_cores=NUM_CORES, num_subcores=NUM_SUBCORES)


@functools.partial(
    pl.kernel,
    out_type=(
        jax.ShapeDtypeStruct((BATCH, 2 * EMB_DIM), jnp.float32),
        jax.ShapeDtypeStruct((BATCH, 2 * EMB_DIM), jnp.float32),
        jax.ShapeDtypeStruct((BATCH, 2 * EMB_DIM), jnp.float32),
    ),
    mesh=_mesh,
    compiler_params=pltpu.CompilerParams(use_tc_tiling_on_sc=True),
    scratch_types=[
        pltpu.VMEM((ROWS_PER_WORKER,), jnp.int32),
        pltpu.VMEM((ROWS_PER_WORKER,), jnp.int32),
        pltpu.VMEM((ROWS_PER_WORKER,), jnp.int32),
        pltpu.VMEM((ROWS_PER_WORKER, 2 * EMB_DIM), jnp.float32),
        pltpu.SemaphoreType.DMA,
        pltpu.SemaphoreType.DMA,
    ],
)
def _sc_gather(ent_p, rel_p, idxh_hbm, idxr_hbm, idxt_hbm,
               h_out, r_out, t_out,
               idxh_v, idxr_v, idxt_v, rows_v, gsem, osem):
    wid = jax.lax.axis_index("s") * NUM_CORES + jax.lax.axis_index("c")
    base = wid * ROWS_PER_WORKER
    jobs = ((idxh_hbm, ent_p, h_out, idxh_v),
            (idxr_hbm, rel_p, r_out, idxr_v),
            (idxt_hbm, ent_p, t_out, idxt_v))
    for idx_hbm, _, _, idx_v in jobs:
        pltpu.sync_copy(idx_hbm.at[pl.ds(base, ROWS_PER_WORKER)], idx_v)
        for k in range(ROWS_PER_WORKER // 16):
            sl = pl.ds(k * 16, 16)
            idx_v[sl] = jax.lax.shift_right_logical(idx_v[sl], 1)
    chunk_jobs = [(t, j) for t in range(len(jobs))
                  for j in range(CHUNKS_PER_WORKER)]
    depth = CHUNKS_PER_WORKER
    n = len(chunk_jobs)
    g = [None] * n
    o = [None] * n

    def _issue_out(k):
        t, j = chunk_jobs[k]
        out = jobs[t][2]
        ssl = pl.ds((k % depth) * CHUNK, CHUNK)
        o[k] = pltpu.async_copy(
            rows_v.at[ssl], out.at[pl.ds(base + j * CHUNK, CHUNK)], osem)

    for k in range(n):
        t, j = chunk_jobs[k]
        tbl, idx_v = jobs[t][1], jobs[t][3]
        if k >= depth:
            o[k - depth].wait()
        g[k] = pltpu.async_copy(
            tbl.at[idx_v.at[pl.ds(j * CHUNK, CHUNK)]],
            rows_v.at[pl.ds((k % depth) * CHUNK, CHUNK)], gsem)
        if k >= 1:
            g[k - 1].wait()
            _issue_out(k - 1)
    g[n - 1].wait()
    _issue_out(n - 1)
    for k in range(n - depth, n):
        o[k].wait()


_NORM_BLOCK = 1024


def _half(x, idx):
    par = (idx & 1) == 1
    return jnp.where(par, x[:, EMB_DIM:], x[:, :EMB_DIM])


def _mxu_t(v):
    eye = jnp.eye(EMB_DIM, dtype=jnp.float32)
    return jax.lax.dot_general(
        eye, v, (((1,), (1,)), ((), ())), preferred_element_type=jnp.float32)


def _norm_body(idxh_ref, h_ref, idxt_ref, t_ref, idxr_ref, r_ref,
               ho_ref, to_ref, ro_ref):
    idxr = idxr_ref[...]
    ro_ref[...] = _mxu_t(_half(r_ref[...], idxr))
    for idx_ref, x_ref, o_ref in ((idxh_ref, h_ref, ho_ref),
                                  (idxt_ref, t_ref, to_ref)):
        idx = idx_ref[...]
        v = _half(x_ref[...], idx)
        keep = idx == NUM_ENTITIES - 1
        norm = jnp.sqrt(jnp.sum(v * v, axis=1, keepdims=True))
        o_ref[...] = _mxu_t(jnp.where(keep, v, v / norm))


def _normalize(idx_h, h_p, idx_t, t_p, idx_r, r_p):
    grid = BATCH // _NORM_BLOCK
    pair_spec = pl.BlockSpec((_NORM_BLOCK, 2 * EMB_DIM), lambda i: (i, 0))
    out_spec = pl.BlockSpec((EMB_DIM, _NORM_BLOCK), lambda i: (0, i))
    idx_spec = pl.BlockSpec((_NORM_BLOCK, 1), lambda i: (i, 0))
    return pl.pallas_call(
        _norm_body,
        grid=(grid,),
        in_specs=[idx_spec, pair_spec, idx_spec, pair_spec,
                  idx_spec, pair_spec],
        out_specs=[out_spec, out_spec, out_spec],
        out_shape=[
            jax.ShapeDtypeStruct((EMB_DIM, BATCH), jnp.float32),
            jax.ShapeDtypeStruct((EMB_DIM, BATCH), jnp.float32),
            jax.ShapeDtypeStruct((EMB_DIM, BATCH), jnp.float32),
        ],
    )(idx_h, h_p, idx_t, t_p, idx_r, r_p)


def kernel(pos_h, pos_r, pos_t, entity_embds, rel_embds):
    ph = pos_h.astype(jnp.int32)
    pr = pos_r.astype(jnp.int32)
    pt = pos_t.astype(jnp.int32)
    ent_p = _repack(jnp.swapaxes(entity_embds, 0, 1), NUM_ENTITIES)
    rel_p = _repack(jnp.swapaxes(rel_embds, 0, 1), NUM_RELATIONS)
    h_p, r_p, t_p = _sc_gather(ent_p, rel_p, ph, pr, pt)
    h_t, t_t, r_t = _normalize(
        ph.reshape(BATCH, 1), h_p, pt.reshape(BATCH, 1), t_p,
        pr.reshape(BATCH, 1), r_p)
    return (jnp.swapaxes(h_t, 0, 1),
            jnp.swapaxes(r_t, 0, 1),
            jnp.swapaxes(t_t, 0, 1))


# speedup vs baseline: 8.2249x; 1.0063x over previous
import functools

import jax
import jax.numpy as jnp
from jax.experimental import pallas as pl
from jax.experimental.pallas import tpu as pltpu
from jax.experimental.pallas import tpu_sc as plsc

NUM_ENTITIES = 1000000
NUM_RELATIONS = 1000
EMB_DIM = 64
BATCH = 16384

NUM_CORES = 2
NUM_SUBCORES = 16
NUM_WORKERS = NUM_CORES * NUM_SUBCORES
ROWS_PER_WORKER = BATCH // NUM_WORKERS
CHUNK = 128
CHUNKS_PER_WORKER = ROWS_PER_WORKER // CHUNK

_REPACK_W = 49152


def _repack_body(x_ref, o_ref):
    sel = jax.lax.broadcasted_iota(jnp.int32, (EMB_DIM, 2 * EMB_DIM), 0)
    tgt = jax.lax.broadcasted_iota(jnp.int32, (EMB_DIM, 2 * EMB_DIM), 1)
    s_even = (tgt == 2 * sel).astype(jnp.float32)
    s_odd = (tgt == 2 * sel + 1).astype(jnp.float32)
    dn = (((1,), (1,)), ((), ()))
    for c in range(_REPACK_W // (2 * EMB_DIM)):
        x = x_ref[:, pl.ds(c * 2 * EMB_DIM, 2 * EMB_DIM)]
        even = jax.lax.dot_general(
            s_even, x, dn, preferred_element_type=jnp.float32)
        odd = jax.lax.dot_general(
            s_odd, x, dn, preferred_element_type=jnp.float32)
        o_ref[pl.ds(c * EMB_DIM, EMB_DIM), :EMB_DIM] = even
        o_ref[pl.ds(c * EMB_DIM, EMB_DIM), EMB_DIM:] = odd


def _repack(tbl_t, n_rows):
    n_packed = (n_rows + 1) // 2
    grid = (n_rows + _REPACK_W - 1) // _REPACK_W
    return pl.pallas_call(
        _repack_body,
        grid=(grid,),
        compiler_params=pltpu.CompilerParams(
            vmem_limit_bytes=100 * 1024 * 1024),
        in_specs=[pl.BlockSpec((EMB_DIM, _REPACK_W), lambda i: (0, i))],
        out_specs=pl.BlockSpec((_REPACK_W // 2, 2 * EMB_DIM), lambda i: (i, 0)),
        out_shape=jax.ShapeDtypeStruct((n_packed, 2 * EMB_DIM), jnp.float32),
    )(tbl_t)


_mesh = plsc.VectorSubcoreMesh(
    core_axis_name="c", subcore_axis_name="s",
    num_cores=NUM_CORES, num_subcores=NUM_SUBCORES)


@functools.partial(
    pl.kernel,
    out_type=(
        jax.ShapeDtypeStruct((BATCH, 2 * EMB_DIM), jnp.float32),
        jax.ShapeDtypeStruct((BATCH, 2 * EMB_DIM), jnp.float32),
        jax.ShapeDtypeStruct((BATCH, 2 * EMB_DIM), jnp.float32),
    ),
    mesh=_mesh,
    compiler_params=pltpu.CompilerParams(use_tc_tiling_on_sc=True),
    scratch_types=[
        pltpu.VMEM((ROWS_PER_WORKER,), jnp.int32),
        pltpu.VMEM((ROWS_PER_WORKER,), jnp.int32),
        pltpu.VMEM((ROWS_PER_WORKER,), jnp.int32),
        pltpu.VMEM((ROWS_PER_WORKER, 2 * EMB_DIM), jnp.float32),
        pltpu.SemaphoreType.DMA,
        pltpu.SemaphoreType.DMA,
    ],
)
def _sc_gather(ent_p, rel_p, idxh_hbm, idxr_hbm, idxt_hbm,
               h_out, r_out, t_out,
               idxh_v, idxr_v, idxt_v, rows_v, gsem, osem):
    wid = jax.lax.axis_index("s") * NUM_CORES + jax.lax.axis_index("c")
    base = wid * ROWS_PER_WORKER
    jobs = ((idxh_hbm, ent_p, h_out, idxh_v),
            (idxr_hbm, rel_p, r_out, idxr_v),
            (idxt_hbm, ent_p, t_out, idxt_v))
    for idx_hbm, _, _, idx_v in jobs:
        pltpu.sync_copy(idx_hbm.at[pl.ds(base, ROWS_PER_WORKER)], idx_v)
        for k in range(ROWS_PER_WORKER // 16):
            sl = pl.ds(k * 16, 16)
            idx_v[sl] = jax.lax.shift_right_logical(idx_v[sl], 1)
    chunk_jobs = [(t, j) for t in range(len(jobs))
                  for j in range(CHUNKS_PER_WORKER)]
    depth = CHUNKS_PER_WORKER
    n = len(chunk_jobs)
    g = [None] * n
    o = [None] * n

    def _issue_out(k):
        t, j = chunk_jobs[k]
        out = jobs[t][2]
        ssl = pl.ds((k % depth) * CHUNK, CHUNK)
        o[k] = pltpu.async_copy(
            rows_v.at[ssl], out.at[pl.ds(base + j * CHUNK, CHUNK)], osem)

    for k in range(n):
        t, j = chunk_jobs[k]
        tbl, idx_v = jobs[t][1], jobs[t][3]
        if k >= depth:
            o[k - depth].wait()
        g[k] = pltpu.async_copy(
            tbl.at[idx_v.at[pl.ds(j * CHUNK, CHUNK)]],
            rows_v.at[pl.ds((k % depth) * CHUNK, CHUNK)], gsem)
        if k >= 1:
            g[k - 1].wait()
            _issue_out(k - 1)
    g[n - 1].wait()
    _issue_out(n - 1)
    for k in range(n - depth, n):
        o[k].wait()


_NORM_BLOCK = 4096


def _half(x, idx):
    par = (idx & 1) == 1
    return jnp.where(par, x[:, EMB_DIM:], x[:, :EMB_DIM])


def _mxu_t(v):
    eye = jnp.eye(EMB_DIM, dtype=jnp.float32)
    return jax.lax.dot_general(
        eye, v, (((1,), (1,)), ((), ())), preferred_element_type=jnp.float32)


def _norm_body(idxh_ref, h_ref, idxt_ref, t_ref, idxr_ref, r_ref,
               ho_ref, to_ref, ro_ref):
    idxr = idxr_ref[...]
    ro_ref[...] = _mxu_t(_half(r_ref[...], idxr))
    for idx_ref, x_ref, o_ref in ((idxh_ref, h_ref, ho_ref),
                                  (idxt_ref, t_ref, to_ref)):
        idx = idx_ref[...]
        v = _half(x_ref[...], idx)
        keep = idx == NUM_ENTITIES - 1
        norm = jnp.sqrt(jnp.sum(v * v, axis=1, keepdims=True))
        o_ref[...] = _mxu_t(jnp.where(keep, v, v / norm))


def _normalize(idx_h, h_p, idx_t, t_p, idx_r, r_p):
    grid = BATCH // _NORM_BLOCK
    pair_spec = pl.BlockSpec((_NORM_BLOCK, 2 * EMB_DIM), lambda i: (i, 0))
    out_spec = pl.BlockSpec((EMB_DIM, _NORM_BLOCK), lambda i: (0, i))
    idx_spec = pl.BlockSpec((_NORM_BLOCK, 1), lambda i: (i, 0))
    return pl.pallas_call(
        _norm_body,
        grid=(grid,),
        in_specs=[idx_spec, pair_spec, idx_spec, pair_spec,
                  idx_spec, pair_spec],
        out_specs=[out_spec, out_spec, out_spec],
        out_shape=[
            jax.ShapeDtypeStruct((EMB_DIM, BATCH), jnp.float32),
            jax.ShapeDtypeStruct((EMB_DIM, BATCH), jnp.float32),
            jax.ShapeDtypeStruct((EMB_DIM, BATCH), jnp.float32),
        ],
    )(idx_h, h_p, idx_t, t_p, idx_r, r_p)


def kernel(pos_h, pos_r, pos_t, entity_embds, rel_embds):
    ph = pos_h.astype(jnp.int32)
    pr = pos_r.astype(jnp.int32)
    pt = pos_t.astype(jnp.int32)
    ent_p = _repack(jnp.swapaxes(entity_embds, 0, 1), NUM_ENTITIES)
    rel_p = _repack(jnp.swapaxes(rel_embds, 0, 1), NUM_RELATIONS)
    h_p, r_p, t_p = _sc_gather(ent_p, rel_p, ph, pr, pt)
    h_t, t_t, r_t = _normalize(
        ph.reshape(BATCH, 1), h_p, pt.reshape(BATCH, 1), t_p,
        pr.reshape(BATCH, 1), r_p)
    return (jnp.swapaxes(h_t, 0, 1),
            jnp.swapaxes(r_t, 0, 1),
            jnp.swapaxes(t_t, 0, 1))
